# baseline (speedup 1.0000x reference)
import functools

import jax
import jax.numpy as jnp
from jax import lax
from jax.experimental import pallas as pl
from jax.experimental.pallas import tpu as pltpu
from jax.experimental.pallas import tpu_sc as plsc

_info = plsc.get_sparse_core_info()
_NC, _NS = _info.num_cores, _info.num_subcores
_NW = _NC * _NS
_L = 16


def _make_gather(batch: int, hist: int, emb_dim: int, nb: int):
    rows_per_w = batch // _NW
    n_chunks = rows_per_w // nb
    assert batch % _NW == 0 and rows_per_w % nb == 0
    pack = 128 // emb_dim
    n_flat = nb * hist
    assert n_flat % _L == 0
    n_groups = n_flat // _L
    mesh = plsc.VectorSubcoreMesh(core_axis_name="c", subcore_axis_name="s")

    @functools.partial(
        pl.kernel,
        mesh=mesh,
        out_type=jax.ShapeDtypeStruct((batch, hist, emb_dim), jnp.float32),
        scratch_types=[
            pltpu.VMEM((n_flat,), jnp.int32),
            pltpu.VMEM((n_flat,), jnp.int32),
            pltpu.VMEM((n_flat, 128), jnp.float32),
            pltpu.VMEM((n_flat, emb_dim), jnp.float32),
            pltpu.SemaphoreType.DMA,
            pltpu.SemaphoreType.DMA,
        ],
        compiler_params=pltpu.CompilerParams(
            use_tc_tiling_on_sc=True, needs_layout_passes=False
        ),
    )
    def gather_kernel(idx_hbm, table_hbm, out_hbm, flat_v, gidx_v, rows_v,
                      out32_v, sem, sem2):
        wid = lax.axis_index("s") * _NC + lax.axis_index("c")
        base = wid * rows_per_w
        lanes = lax.iota(jnp.int32, _L)

        def body(i, carry):
            r0 = base + i * nb
            pltpu.sync_copy(idx_hbm.at[pl.ds(r0 * hist, n_flat)], flat_v)

            def to_gidx(g, c):
                v = flat_v[pl.ds(g * _L, _L)]
                gidx_v[pl.ds(g * _L, _L)] = lax.shift_right_logical(
                    v, jnp.int32(2)
                )
                return c

            lax.fori_loop(0, n_groups, to_gidx, 0)
            pltpu.async_copy(table_hbm.at[gidx_v], rows_v, sem).wait()

            def select(g, c):
                k0 = g * _L
                kvec = lanes + k0
                sub = flat_v[pl.ds(k0, _L)] & jnp.int32(pack - 1)
                col0 = sub * jnp.int32(emb_dim)
                for d in range(emb_dim):
                    vals = plsc.load_gather(
                        rows_v, [kvec, col0 + jnp.int32(d)]
                    )
                    plsc.store_scatter(
                        out32_v,
                        [kvec, jnp.full((_L,), d, jnp.int32)],
                        vals,
                    )
                return c

            lax.fori_loop(0, n_groups, select, 0)
            handles = [
                pltpu.async_copy(
                    out32_v.at[pl.ds(r * hist, hist), :],
                    out_hbm.at[r0 + r],
                    sem2,
                )
                for r in range(nb)
            ]
            for h in handles:
                h.wait()
            return carry

        lax.fori_loop(0, n_chunks, body, 0)

    return gather_kernel


def kernel(inputs, table):
    batch, hist = inputs.shape
    n_vocab, emb_dim = table.shape
    idx_flat = inputs.reshape(-1)
    table_c = table.reshape(n_vocab * emb_dim // 128, 128)
    return _make_gather(batch, hist, emb_dim, nb=8)(idx_flat, table_c)


# speedup vs baseline: 2.0403x; 2.0403x over previous
import functools

import jax
import jax.numpy as jnp
from jax import lax
from jax.experimental import pallas as pl
from jax.experimental.pallas import tpu as pltpu
from jax.experimental.pallas import tpu_sc as plsc

_info = plsc.get_sparse_core_info()
_NC, _NS = _info.num_cores, _info.num_subcores
_NW = _NC * _NS


def _make_gather(batch: int, hist: int, emb_dim: int, nb: int):
    rows_per_w = batch // _NW
    n_chunks = rows_per_w // nb
    assert batch % _NW == 0 and rows_per_w % nb == 0
    n_flat = nb * hist
    mesh = plsc.VectorSubcoreMesh(core_axis_name="c", subcore_axis_name="s")

    @functools.partial(
        pl.kernel,
        mesh=mesh,
        out_type=jax.ShapeDtypeStruct((batch, hist, emb_dim), jnp.float32),
        scratch_types=[
            pltpu.VMEM((n_flat,), jnp.int32),
            pltpu.VMEM((n_flat, emb_dim), jnp.float32),
            pltpu.SemaphoreType.DMA,
            pltpu.SemaphoreType.DMA,
        ],
        compiler_params=pltpu.CompilerParams(use_tc_tiling_on_sc=False),
    )
    def gather_kernel(idx_hbm, table_hbm, out_hbm, flat_v, rows_v, sem, sem2):
        wid = lax.axis_index("s") * _NC + lax.axis_index("c")
        base = wid * rows_per_w

        def body(i, carry):
            r0 = base + i * nb
            pltpu.sync_copy(idx_hbm.at[pl.ds(r0 * hist, n_flat)], flat_v)
            pltpu.async_copy(table_hbm.at[flat_v], rows_v, sem).wait()
            handles = [
                pltpu.async_copy(
                    rows_v.at[pl.ds(r * hist, hist), :],
                    out_hbm.at[r0 + r],
                    sem2,
                )
                for r in range(nb)
            ]
            for h in handles:
                h.wait()
            return carry

        lax.fori_loop(0, n_chunks, body, 0)

    return gather_kernel


def kernel(inputs, table):
    batch, hist = inputs.shape
    n_vocab, emb_dim = table.shape
    idx_flat = jnp.minimum(inputs.reshape(-1), jnp.int32(n_vocab - 1))
    return _make_gather(batch, hist, emb_dim, nb=64)(idx_flat, table)
